# baseline (speedup 1.0000x reference)
import functools

import jax
import jax.numpy as jnp
from jax import lax
from jax.experimental import pallas as pl
from jax.experimental.pallas import tpu as pltpu
from jax.experimental.pallas import tpu_sc as plsc

N = 10000
NP = 10240
D = 128
G = 64
E = 320000
EPS = 1e-5
NC = 2
NS = 16
CHUNK = 50
CPT = 200
RPT = NP // NS
R = 1024
NBUF = 5
GB = 8
NSL = 3
NGRP = CPT // GB

_mesh = plsc.VectorSubcoreMesh(core_axis_name="c", subcore_axis_name="s")


@functools.partial(
    pl.kernel,
    out_type=jax.ShapeDtypeStruct((NC, NP), jnp.float32),
    mesh=_mesh,
    scratch_types=[
        pltpu.VMEM_SHARED((NP,), jnp.float32),
        pltpu.VMEM((NGRP, GB, CHUNK), jnp.int32),
        pltpu.VMEM((CHUNK,), jnp.float32),
        pltpu.VMEM((RPT,), jnp.float32),
        pltpu.SemaphoreType.DMA,
    ],
)
def _sc_degree(dstr, out, dacc, dst_v, ones_v, z_v, sem):
    c = lax.axis_index("c")
    s = lax.axis_index("s")
    r0 = s * RPT
    pltpu.sync_copy(dstr.at[c, s], dst_v)
    for i in range(CHUNK // 16):
        ones_v[pl.ds(i * 16, 16)] = jnp.ones((16,), jnp.float32)
    if CHUNK % 16:
        ones_v[pl.ds(CHUNK - 16, 16)] = jnp.ones((16,), jnp.float32)

    def zfill(i, carry):
        z_v[pl.ds(i * 16, 16)] = jnp.zeros((16,), jnp.float32)
        return carry

    lax.fori_loop(0, RPT // 16, zfill, 0)
    pltpu.sync_copy(z_v, dacc.at[pl.ds(r0, RPT)])
    plsc.subcore_barrier()

    def group(g, carry):
        for b in range(GB):
            pltpu.async_copy(ones_v, dacc.at[dst_v.at[g, b]], sem, add=True)
        for b in range(GB):
            pltpu.make_async_copy(ones_v, dacc.at[dst_v.at[g, b]], sem).wait()
        return carry

    lax.fori_loop(0, NGRP, group, 0)
    plsc.subcore_barrier()
    pltpu.sync_copy(dacc.at[pl.ds(r0, RPT)], out.at[c, pl.ds(r0, RPT)])


@functools.partial(
    pl.kernel,
    out_type=jax.ShapeDtypeStruct((NC, NP, D), jnp.float32),
    mesh=_mesh,
    scratch_types=[
        pltpu.VMEM_SHARED((NP, D), jnp.float32),
        pltpu.VMEM((NSL, GB, CHUNK), jnp.int32),
        pltpu.VMEM((NSL, GB, CHUNK), jnp.int32),
        pltpu.VMEM((NBUF, CHUNK, D), jnp.float32),
        pltpu.SemaphoreType.DMA,
        pltpu.SemaphoreType.DMA,
        pltpu.SemaphoreType.DMA,
    ],
)
def _sc_aggregate(hp, srcr, dstr, out, acc, sidx, didx, bufs, gsem, ssem, isem):
    c = lax.axis_index("c")
    s = lax.axis_index("s")
    r0 = s * RPT
    pltpu.sync_copy(srcr.at[c, s, 0], sidx.at[0])
    pltpu.sync_copy(dstr.at[c, s, 0], didx.at[0])
    if NGRP > 1:
        pltpu.async_copy(srcr.at[c, s, 1], sidx.at[1], isem)
        pltpu.async_copy(dstr.at[c, s, 1], didx.at[1], isem)
    pltpu.sync_copy(hp.at[pl.ds(r0, RPT)], acc.at[pl.ds(r0, RPT)])
    plsc.subcore_barrier()

    for b in range(NBUF - 1):
        pltpu.async_copy(hp.at[sidx.at[0, b]], bufs.at[b], gsem)

    def step(j, carry):
        g = j // GB
        p = j % GB
        sl = g % NSL
        b = j % NBUF
        jm = j - 1
        slm = (jm // GB) % NSL
        pm = jm % GB
        bm = jm % NBUF
        j2 = j + (NBUF - 1)
        sl2 = (j2 // GB) % NSL
        p2 = j2 % GB
        b2 = j2 % NBUF

        pltpu.make_async_copy(hp.at[sidx.at[sl, p]], bufs.at[b], gsem).wait()
        pltpu.async_copy(bufs.at[b], acc.at[didx.at[sl, p]], ssem, add=True)

        @pl.when(j > 0)
        def _wait_prev_scatter():
            pltpu.make_async_copy(
                bufs.at[bm], acc.at[didx.at[slm, pm]], ssem
            ).wait()

        @pl.when(jnp.logical_and(p == 0, g + 2 < NGRP))
        def _prefetch_idx():
            pltpu.async_copy(srcr.at[c, s, g + 2], sidx.at[(g + 2) % NSL], isem)
            pltpu.async_copy(dstr.at[c, s, g + 2], didx.at[(g + 2) % NSL], isem)

        @pl.when(jnp.logical_and(p == 1, g + 1 < NGRP))
        def _wait_idx():
            pltpu.make_async_copy(
                srcr.at[c, s, g + 1], sidx.at[(g + 1) % NSL], isem
            ).wait()
            pltpu.make_async_copy(
                dstr.at[c, s, g + 1], didx.at[(g + 1) % NSL], isem
            ).wait()

        @pl.when(j2 < CPT)
        def _next_gather():
            pltpu.async_copy(hp.at[sidx.at[sl2, p2]], bufs.at[b2], gsem)

        return carry

    lax.fori_loop(0, CPT, step, 0)
    pltpu.make_async_copy(
        bufs.at[(CPT - 1) % NBUF],
        acc.at[didx.at[((CPT - 1) // GB) % NSL, (CPT - 1) % GB]],
        ssem,
    ).wait()
    plsc.subcore_barrier()
    pltpu.sync_copy(acc.at[pl.ds(r0, RPT)], out.at[c, pl.ds(r0, RPT)])


def _dinv_col(deg):
    dg = deg[0:1, :] + deg[1:2, :] + 1.0
    return jnp.transpose(lax.rsqrt(dg))


def _tc1_body(x, w1, deg, hp1):
    t0 = jnp.dot(x[...], w1[...], preferred_element_type=jnp.float32)
    di = _dinv_col(deg[...])
    hp1[0:N, :] = t0 * di[0:N, :]
    hp1[N:NP, :] = jnp.zeros((NP - N, D), jnp.float32)


def _tc_scale_in(x, w1, deg):
    return pl.pallas_call(
        _tc1_body,
        out_shape=jax.ShapeDtypeStruct((NP, D), jnp.float32),
    )(x, w1, deg)


def _tc_mid_body(agg, hp1, deg, b1, gamma, beta, w2, hp2):
    di = _dinv_col(deg[...])
    h = (agg[0] + agg[1] - hp1[...]) * di + b1[...]
    rows = lax.broadcasted_iota(jnp.int32, (NP, 1), 0)
    m = (rows < N).astype(jnp.float32)
    hm = h * m
    mean = jnp.sum(hm, axis=0, keepdims=True) / N
    var = jnp.sum(hm * h, axis=0, keepdims=True) / N - mean * mean
    inv = lax.rsqrt(var + EPS)
    hb = (h - mean) * inv * gamma[...] + beta[...]
    hb = jnp.maximum(hb, 0.0)
    hp2[...] = (
        jnp.dot(hb, w2[...], preferred_element_type=jnp.float32) * di
    )


def _tc_mid(agg1, hp1, deg, b1, gamma, beta, w2):
    return pl.pallas_call(
        _tc_mid_body,
        out_shape=jax.ShapeDtypeStruct((NP, D), jnp.float32),
    )(agg1, hp1, deg, b1, gamma, beta, w2)


def _tc4_body(agg, hp2, deg, b2, batch_row, linw, linb, out):
    h2 = (agg[0] + agg[1] - hp2[...]) * _dinv_col(deg[...]) + b2[...]
    bcol = jnp.transpose(batch_row[...])
    oh = (bcol == lax.broadcasted_iota(jnp.int32, (N, G), 1)).astype(
        jnp.float32
    )
    sums = lax.dot_general(
        oh, h2[0:N, :], (((0,), (0,)), ((), ())),
        preferred_element_type=jnp.float32,
    )
    cnts = lax.dot_general(
        oh,
        jnp.ones((N, 1), jnp.float32),
        (((0,), (0,)), ((), ())),
        preferred_element_type=jnp.float32,
    )
    pooled = sums / jnp.maximum(cnts, 1.0)
    out[...] = (
        jnp.dot(pooled, linw[...], preferred_element_type=jnp.float32)
        + linb[...]
    )


def _tc_final(agg2, hp2, deg, b2, batch_row, lin_w, lin_b):
    return pl.pallas_call(
        _tc4_body,
        out_shape=jax.ShapeDtypeStruct((G, D), jnp.float32),
    )(agg2, hp2, deg, b2, batch_row, lin_w, lin_b)


def kernel(x, edge_index, batch, W1, b1, W2, b2, bn_gamma, bn_beta, lin_W, lin_b):
    srcp = edge_index[0].astype(jnp.int32).reshape(NC, NS, NGRP, GB, CHUNK)
    dstp = edge_index[1].astype(jnp.int32).reshape(NC, NS, NGRP, GB, CHUNK)
    batch_row = batch.astype(jnp.int32).reshape(1, N)

    deg = _sc_degree(dstp)
    hp1 = _tc_scale_in(x, W1, deg)
    agg1 = _sc_aggregate(hp1, srcp, dstp)
    hp2 = _tc_mid(agg1, hp1, deg, b1, bn_gamma, bn_beta, W2)
    agg2 = _sc_aggregate(hp2, srcp, dstp)
    return _tc_final(agg2, hp2, deg, b2, batch_row, lin_W, lin_b)


# speedup vs baseline: 1.0221x; 1.0221x over previous
import functools

import jax
import jax.numpy as jnp
from jax import lax
from jax.experimental import pallas as pl
from jax.experimental.pallas import tpu as pltpu
from jax.experimental.pallas import tpu_sc as plsc

N = 10000
NP = 10240
D = 128
G = 64
E = 320000
EPS = 1e-5
NC = 2
NS = 16
CHUNK = 80
CPT = 125
RPT = NP // NS
R = 1024
NBUF = 4
GB = 5
NSL = 3
NGRP = CPT // GB

_mesh = plsc.VectorSubcoreMesh(core_axis_name="c", subcore_axis_name="s")


@functools.partial(
    pl.kernel,
    out_type=jax.ShapeDtypeStruct((NC, NP), jnp.float32),
    mesh=_mesh,
    scratch_types=[
        pltpu.VMEM_SHARED((NP,), jnp.float32),
        pltpu.VMEM((NGRP, GB, CHUNK), jnp.int32),
        pltpu.VMEM((CHUNK,), jnp.float32),
        pltpu.VMEM((RPT,), jnp.float32),
        pltpu.SemaphoreType.DMA,
    ],
)
def _sc_degree(dstr, out, dacc, dst_v, ones_v, z_v, sem):
    c = lax.axis_index("c")
    s = lax.axis_index("s")
    r0 = s * RPT
    pltpu.sync_copy(dstr.at[c, s], dst_v)
    for i in range(CHUNK // 16):
        ones_v[pl.ds(i * 16, 16)] = jnp.ones((16,), jnp.float32)
    if CHUNK % 16:
        ones_v[pl.ds(CHUNK - 16, 16)] = jnp.ones((16,), jnp.float32)

    def zfill(i, carry):
        z_v[pl.ds(i * 16, 16)] = jnp.zeros((16,), jnp.float32)
        return carry

    lax.fori_loop(0, RPT // 16, zfill, 0)
    pltpu.sync_copy(z_v, dacc.at[pl.ds(r0, RPT)])
    plsc.subcore_barrier()

    def group(g, carry):
        for b in range(GB):
            pltpu.async_copy(ones_v, dacc.at[dst_v.at[g, b]], sem, add=True)
        for b in range(GB):
            pltpu.make_async_copy(ones_v, dacc.at[dst_v.at[g, b]], sem).wait()
        return carry

    lax.fori_loop(0, NGRP, group, 0)
    plsc.subcore_barrier()
    pltpu.sync_copy(dacc.at[pl.ds(r0, RPT)], out.at[c, pl.ds(r0, RPT)])


@functools.partial(
    pl.kernel,
    out_type=jax.ShapeDtypeStruct((NC, NP, D), jnp.float32),
    mesh=_mesh,
    scratch_types=[
        pltpu.VMEM_SHARED((NP, D), jnp.float32),
        pltpu.VMEM((NSL, GB, CHUNK), jnp.int32),
        pltpu.VMEM((NSL, GB, CHUNK), jnp.int32),
        pltpu.VMEM((NBUF, CHUNK, D), jnp.float32),
        pltpu.SemaphoreType.DMA,
        pltpu.SemaphoreType.DMA,
        pltpu.SemaphoreType.DMA,
    ],
)
def _sc_aggregate(hp, srcr, dstr, out, acc, sidx, didx, bufs, gsem, ssem, isem):
    c = lax.axis_index("c")
    s = lax.axis_index("s")
    r0 = s * RPT
    pltpu.sync_copy(srcr.at[c, s, 0], sidx.at[0])
    pltpu.sync_copy(dstr.at[c, s, 0], didx.at[0])
    if NGRP > 1:
        pltpu.async_copy(srcr.at[c, s, 1], sidx.at[1], isem)
        pltpu.async_copy(dstr.at[c, s, 1], didx.at[1], isem)
    pltpu.sync_copy(hp.at[pl.ds(r0, RPT)], acc.at[pl.ds(r0, RPT)])
    plsc.subcore_barrier()

    for b in range(NBUF - 1):
        pltpu.async_copy(hp.at[sidx.at[0, b]], bufs.at[b], gsem)

    def step(j, carry):
        g = j // GB
        p = j % GB
        sl = g % NSL
        b = j % NBUF
        jm = j - 1
        slm = (jm // GB) % NSL
        pm = jm % GB
        bm = jm % NBUF
        j2 = j + (NBUF - 1)
        sl2 = (j2 // GB) % NSL
        p2 = j2 % GB
        b2 = j2 % NBUF

        pltpu.make_async_copy(hp.at[sidx.at[sl, p]], bufs.at[b], gsem).wait()
        pltpu.async_copy(bufs.at[b], acc.at[didx.at[sl, p]], ssem, add=True)

        @pl.when(j > 0)
        def _wait_prev_scatter():
            pltpu.make_async_copy(
                bufs.at[bm], acc.at[didx.at[slm, pm]], ssem
            ).wait()

        @pl.when(jnp.logical_and(p == 0, g + 2 < NGRP))
        def _prefetch_idx():
            pltpu.async_copy(srcr.at[c, s, g + 2], sidx.at[(g + 2) % NSL], isem)
            pltpu.async_copy(dstr.at[c, s, g + 2], didx.at[(g + 2) % NSL], isem)

        @pl.when(jnp.logical_and(p == 1, g + 1 < NGRP))
        def _wait_idx():
            pltpu.make_async_copy(
                srcr.at[c, s, g + 1], sidx.at[(g + 1) % NSL], isem
            ).wait()
            pltpu.make_async_copy(
                dstr.at[c, s, g + 1], didx.at[(g + 1) % NSL], isem
            ).wait()

        @pl.when(j2 < CPT)
        def _next_gather():
            pltpu.async_copy(hp.at[sidx.at[sl2, p2]], bufs.at[b2], gsem)

        return carry

    lax.fori_loop(0, CPT, step, 0)
    pltpu.make_async_copy(
        bufs.at[(CPT - 1) % NBUF],
        acc.at[didx.at[((CPT - 1) // GB) % NSL, (CPT - 1) % GB]],
        ssem,
    ).wait()
    plsc.subcore_barrier()
    pltpu.sync_copy(acc.at[pl.ds(r0, RPT)], out.at[c, pl.ds(r0, RPT)])


def _dinv_col(deg):
    dg = deg[0:1, :] + deg[1:2, :] + 1.0
    return jnp.transpose(lax.rsqrt(dg))


def _tc1_body(x, w1, deg, hp1):
    t0 = jnp.dot(x[...], w1[...], preferred_element_type=jnp.float32)
    di = _dinv_col(deg[...])
    hp1[0:N, :] = t0 * di[0:N, :]
    hp1[N:NP, :] = jnp.zeros((NP - N, D), jnp.float32)


def _tc_scale_in(x, w1, deg):
    return pl.pallas_call(
        _tc1_body,
        out_shape=jax.ShapeDtypeStruct((NP, D), jnp.float32),
    )(x, w1, deg)


def _tc_mid_body(agg, hp1, deg, b1, gamma, beta, w2, hp2):
    di = _dinv_col(deg[...])
    h = (agg[0] + agg[1] - hp1[...]) * di + b1[...]
    rows = lax.broadcasted_iota(jnp.int32, (NP, 1), 0)
    m = (rows < N).astype(jnp.float32)
    hm = h * m
    mean = jnp.sum(hm, axis=0, keepdims=True) / N
    var = jnp.sum(hm * h, axis=0, keepdims=True) / N - mean * mean
    inv = lax.rsqrt(var + EPS)
    hb = (h - mean) * inv * gamma[...] + beta[...]
    hb = jnp.maximum(hb, 0.0)
    hp2[...] = (
        jnp.dot(hb, w2[...], preferred_element_type=jnp.float32) * di
    )


def _tc_mid(agg1, hp1, deg, b1, gamma, beta, w2):
    return pl.pallas_call(
        _tc_mid_body,
        out_shape=jax.ShapeDtypeStruct((NP, D), jnp.float32),
    )(agg1, hp1, deg, b1, gamma, beta, w2)


def _tc4_body(agg, hp2, deg, b2, batch_row, linw, linb, out):
    h2 = (agg[0] + agg[1] - hp2[...]) * _dinv_col(deg[...]) + b2[...]
    bcol = jnp.transpose(batch_row[...])
    oh = (bcol == lax.broadcasted_iota(jnp.int32, (N, G), 1)).astype(
        jnp.float32
    )
    sums = lax.dot_general(
        oh, h2[0:N, :], (((0,), (0,)), ((), ())),
        preferred_element_type=jnp.float32,
    )
    cnts = lax.dot_general(
        oh,
        jnp.ones((N, 1), jnp.float32),
        (((0,), (0,)), ((), ())),
        preferred_element_type=jnp.float32,
    )
    pooled = sums / jnp.maximum(cnts, 1.0)
    out[...] = (
        jnp.dot(pooled, linw[...], preferred_element_type=jnp.float32)
        + linb[...]
    )


def _tc_final(agg2, hp2, deg, b2, batch_row, lin_w, lin_b):
    return pl.pallas_call(
        _tc4_body,
        out_shape=jax.ShapeDtypeStruct((G, D), jnp.float32),
    )(agg2, hp2, deg, b2, batch_row, lin_w, lin_b)


def kernel(x, edge_index, batch, W1, b1, W2, b2, bn_gamma, bn_beta, lin_W, lin_b):
    srcp = edge_index[0].astype(jnp.int32).reshape(NC, NS, NGRP, GB, CHUNK)
    dstp = edge_index[1].astype(jnp.int32).reshape(NC, NS, NGRP, GB, CHUNK)
    batch_row = batch.astype(jnp.int32).reshape(1, N)

    deg = _sc_degree(dstp)
    hp1 = _tc_scale_in(x, W1, deg)
    agg1 = _sc_aggregate(hp1, srcp, dstp)
    hp2 = _tc_mid(agg1, hp1, deg, b1, bn_gamma, bn_beta, W2)
    agg2 = _sc_aggregate(hp2, srcp, dstp)
    return _tc_final(agg2, hp2, deg, b2, batch_row, lin_W, lin_b)
